# baseline (speedup 1.0000x reference)
import functools

import jax
import jax.numpy as jnp
from jax import lax
from jax.experimental import pallas as pl
from jax.experimental.pallas import tpu as pltpu
from jax.experimental.pallas import tpu_sc as plsc

N = 10000
E = 320000
D_IN = 128
HID = 64
CLS = 64
K = 10
ALPHA = 0.1

NTILE = 16
NP = 10240
RPT = NP // NTILE
SINK = NP
GROWS = NP + 16
EC = 128
NCHUNK = 158
EPT = NCHUNK * EC
EPAD = NTILE * EPT
QCH = RPT // EC
UEC = 64
UQCH = RPT // UEC


def _mesh():
    return plsc.VectorSubcoreMesh(core_axis_name="c", subcore_axis_name="s",
                                  num_cores=1)


_SC_PARAMS = pltpu.CompilerParams(use_tc_tiling_on_sc=False)


@functools.partial(
    pl.kernel,
    mesh=_mesh(),
    out_type=jax.ShapeDtypeStruct((NP, 16), jnp.float32),
    scratch_types=[
        pltpu.VMEM((NCHUNK, EC), jnp.int32),
        pltpu.VMEM((EC, 16), jnp.float32),
        pltpu.VMEM((EC, 16), jnp.float32),
        pltpu.VMEM_SHARED((GROWS, 16), jnp.float32),
    ],
    compiler_params=_SC_PARAMS,
)
def _deg_kernel(dst_hbm, deg_out, didx, ones, zb, acc):
    wid = lax.axis_index("s")
    r0 = wid * RPT

    pltpu.sync_copy(dst_hbm.at[wid], didx)

    def fill_body(r, _):
        ones[r, pl.ds(0, 16)] = jnp.full((16,), 1.0, jnp.float32)
        zb[r, pl.ds(0, 16)] = jnp.zeros((16,), jnp.float32)
        return _
    lax.fori_loop(0, EC, fill_body, None)

    def zero_body(q, _):
        pltpu.sync_copy(zb, acc.at[pl.ds(r0 + q * EC, EC)])
        return _
    lax.fori_loop(0, QCH, zero_body, None)

    @pl.when(wid == 0)
    def _():
        pltpu.sync_copy(zb.at[pl.ds(0, 16)], acc.at[pl.ds(SINK, 16)])

    plsc.subcore_barrier()

    def chunk_body(c, _):
        pltpu.sync_copy(ones, acc.at[didx.at[c]], add=True)
        return _
    lax.fori_loop(0, NCHUNK, chunk_body, None)

    plsc.subcore_barrier()

    pltpu.sync_copy(acc.at[pl.ds(r0, RPT)], deg_out.at[pl.ds(r0, RPT)])


def _elu(x):
    return jnp.where(x > 0, x, jnp.exp(jnp.minimum(x, 0.0)) - 1.0)


def _make_prep(d_in, apply_elu):
    def body(x_ref, w_ref, b_ref, deg_ref, g0_ref, a2_ref, b2_ref, a1_ref,
             b1_ref):
        h = jnp.dot(x_ref[...], w_ref[...],
                    preferred_element_type=jnp.float32)
        h = h + jnp.broadcast_to(b_ref[0:1, :], h.shape)
        if apply_elu:
            h = _elu(h)
        deg = deg_ref[...][:, 0:1]
        nb = jax.lax.rsqrt(jnp.clip(deg, 1.0, None))
        nb = jnp.broadcast_to(nb, h.shape)
        g0_ref[...] = nb * h
        a2_ref[...] = (1.0 - ALPHA) * nb * nb
        b2_ref[...] = ALPHA * nb * h
        a1_ref[...] = (1.0 - ALPHA) * nb
        b1_ref[...] = ALPHA * h
    rows = 1024
    grid = NP // rows
    out_sds = jax.ShapeDtypeStruct((NP, HID), jnp.float32)
    out_spec = pl.BlockSpec((rows, HID), lambda i: (i, 0))
    return pl.pallas_call(
        body,
        grid=(grid,),
        in_specs=[
            pl.BlockSpec((rows, d_in), lambda i: (i, 0)),
            pl.BlockSpec((d_in, HID), lambda i: (0, 0)),
            pl.BlockSpec((8, HID), lambda i: (0, 0)),
            pl.BlockSpec((rows, 16), lambda i: (i, 0)),
        ],
        out_specs=[out_spec] * 5,
        out_shape=[out_sds] * 5,
    )


def _make_appnp(final_elu):
    @functools.partial(
        pl.kernel,
        mesh=_mesh(),
        out_type=[
            jax.ShapeDtypeStruct((NP, HID), jnp.float32),
            jax.ShapeDtypeStruct((GROWS, HID), jnp.float32),
        ],
        scratch_types=[
            pltpu.VMEM((NCHUNK, EC), jnp.int32),
            pltpu.VMEM((NCHUNK, EC), jnp.int32),
            pltpu.VMEM((4, EC, HID), jnp.float32),
            pltpu.VMEM((UEC, HID), jnp.float32),
            pltpu.VMEM((UEC, HID), jnp.float32),
            pltpu.VMEM((UEC, HID), jnp.float32),
            pltpu.VMEM_SHARED((GROWS, HID), jnp.float32),
            pltpu.SemaphoreType.DMA((4,)),
            pltpu.SemaphoreType.DMA((4,)),
        ],
        compiler_params=_SC_PARAMS,
    )
    def appnp(g0_hbm, a2_hbm, b2_hbm, a1_hbm, b1_hbm, src_hbm, dst_hbm,
              out_hbm, g_hbm, sidx, didx, rows, ab, bb, zb, acc, gsem, ssem):
        wid = lax.axis_index("s")
        r0 = wid * RPT

        pltpu.sync_copy(src_hbm.at[wid], sidx)
        pltpu.sync_copy(dst_hbm.at[wid], didx)

        def zfill(r, _):
            for j in range(HID // 16):
                zb[r, pl.ds(16 * j, 16)] = jnp.zeros((16,), jnp.float32)
            return _
        lax.fori_loop(0, UEC, zfill, None)

        def stage(q, _):
            sl = pl.ds(r0 + q * UEC, UEC)
            stg = rows.at[0, pl.ds(0, UEC)]
            pltpu.sync_copy(g0_hbm.at[sl], stg)
            pltpu.sync_copy(stg, g_hbm.at[sl])
            pltpu.sync_copy(zb, acc.at[sl])
            return _
        lax.fori_loop(0, UQCH, stage, None)

        @pl.when(wid == 0)
        def _():
            pltpu.sync_copy(zb.at[pl.ds(0, 16)], g_hbm.at[pl.ds(SINK, 16)])
            pltpu.sync_copy(zb.at[pl.ds(0, 16)], acc.at[pl.ds(SINK, 16)])

        plsc.subcore_barrier()

        def gather_start(c, b):
            pltpu.async_copy(g_hbm.at[sidx.at[c]], rows.at[b], gsem.at[b])

        def gather_wait(c, b):
            pltpu.make_async_copy(g_hbm.at[sidx.at[c]], rows.at[b],
                                  gsem.at[b]).wait()

        def scatter_start(c, b):
            pltpu.async_copy(rows.at[b], acc.at[didx.at[c]], ssem.at[b],
                             add=True)

        def scatter_wait(b):
            pltpu.make_async_copy(rows.at[b], acc.at[didx.at[0]],
                                  ssem.at[b]).wait()

        def edge_phase():
            gather_start(0, 0)
            gather_start(1, 1)
            gather_start(2, 2)

            def chunk_body(c, _):
                for b in range(4):
                    @pl.when((c % 4) == b)
                    def _():
                        gather_wait(c, b)
                        scatter_start(c, b)
                        bn = (b + 3) % 4

                        @pl.when(c >= 1)
                        def _():
                            scatter_wait(bn)

                        @pl.when(c < NCHUNK - 3)
                        def _():
                            gather_start(c + 3, bn)
                return _
            lax.fori_loop(0, NCHUNK, chunk_body, None)
            scatter_wait((NCHUNK - 1) % 4)

        def update_phase(a_hbm, b_hbm, dst, elu):
            accv = rows.at[0, pl.ds(0, UEC)]
            gout = rows.at[1, pl.ds(0, UEC)]

            def upd(q, _):
                sl = pl.ds(r0 + q * UEC, UEC)
                pltpu.sync_copy(acc.at[sl], accv)
                pltpu.sync_copy(zb, acc.at[sl])
                pltpu.sync_copy(a_hbm.at[sl], ab)
                pltpu.sync_copy(b_hbm.at[sl], bb)

                def rowupd(r, _):
                    for j in range(HID // 16):
                        cs = pl.ds(16 * j, 16)
                        v = ab[r, cs] * rows[0, r, cs] + bb[r, cs]
                        if elu:
                            v = jnp.where(
                                v > 0,
                                v,
                                jnp.exp(jnp.minimum(v, 0.0)) - 1.0)
                        rows[1, r, cs] = v
                    return _
                lax.fori_loop(0, UEC, rowupd, None)
                pltpu.sync_copy(gout, dst.at[sl])
                return _
            lax.fori_loop(0, UQCH, upd, None)

        def iter_body(it, _):
            edge_phase()
            plsc.subcore_barrier()
            update_phase(a2_hbm, b2_hbm, g_hbm, False)
            plsc.subcore_barrier()
            return _
        lax.fori_loop(0, K - 1, iter_body, None)

        edge_phase()
        plsc.subcore_barrier()
        update_phase(a1_hbm, b1_hbm, out_hbm, final_elu)

    return appnp


_appnp_plain = _make_appnp(False)
_appnp_elu = _make_appnp(True)


def kernel(features, edge_index, order_attn, W1, b1, W2, b2):
    del order_attn

    f32 = jnp.float32
    feats = jnp.pad(features.astype(f32), ((0, NP - N), (0, 0)))

    src = edge_index[0].astype(jnp.int32)
    dst = edge_index[1].astype(jnp.int32)
    pad = jnp.full((EPAD - E,), SINK, jnp.int32)
    src3 = jnp.concatenate([src, pad]).reshape(NTILE, NCHUNK, EC)
    dst3 = jnp.concatenate([dst, pad]).reshape(NTILE, NCHUNK, EC)

    deg = _deg_kernel(dst3)

    b1b = jnp.broadcast_to(b1.astype(f32)[None, :], (8, HID))
    b2b = jnp.broadcast_to(b2.astype(f32)[None, :], (8, CLS))

    prep1 = _make_prep(D_IN, False)
    g0, a2, bcoef2, a1, bcoef1 = prep1(feats, W1.astype(f32), b1b, deg)
    x1, _ = _appnp_plain(g0, a2, bcoef2, a1, bcoef1, src3, dst3)

    prep2 = _make_prep(HID, True)
    g0b, a2b, bcoef2b, a1b, bcoef1b = prep2(x1, W2.astype(f32), b2b, deg)
    x2, _ = _appnp_elu(g0b, a2b, bcoef2b, a1b, bcoef1b, src3, dst3)

    return x2[:N]


# speedup vs baseline: 1.1563x; 1.1563x over previous
import functools

import jax
import jax.numpy as jnp
from jax import lax
from jax.experimental import pallas as pl
from jax.experimental.pallas import tpu as pltpu
from jax.experimental.pallas import tpu_sc as plsc

N = 10000
E = 320000
D_IN = 128
HID = 64
CLS = 64
K = 10
ALPHA = 0.1

NTILE = 16
NP = 10240
RPT = NP // NTILE
SINK = NP
GROWS = NP + 16
EC = 128
NCHUNK = 158
EPT = NCHUNK * EC
EPAD = NTILE * EPT
QCH = RPT // EC
UEC = 64
UQCH = RPT // UEC


def _mesh():
    return plsc.VectorSubcoreMesh(core_axis_name="c", subcore_axis_name="s",
                                  num_cores=1)


_SC_PARAMS = pltpu.CompilerParams(use_tc_tiling_on_sc=False)


@functools.partial(
    pl.kernel,
    mesh=_mesh(),
    out_type=jax.ShapeDtypeStruct((NP, 16), jnp.float32),
    scratch_types=[
        pltpu.VMEM((NCHUNK, EC), jnp.int32),
        pltpu.VMEM((EC, 16), jnp.float32),
        pltpu.VMEM((EC, 16), jnp.float32),
        pltpu.VMEM_SHARED((GROWS, 16), jnp.float32),
    ],
    compiler_params=_SC_PARAMS,
)
def _deg_kernel(dst_hbm, deg_out, didx, ones, zb, acc):
    wid = lax.axis_index("s")
    r0 = wid * RPT

    pltpu.sync_copy(dst_hbm.at[wid], didx)

    def fill_body(r, _):
        ones[r, pl.ds(0, 16)] = jnp.full((16,), 1.0, jnp.float32)
        zb[r, pl.ds(0, 16)] = jnp.zeros((16,), jnp.float32)
        return _
    lax.fori_loop(0, EC, fill_body, None)

    def zero_body(q, _):
        pltpu.sync_copy(zb, acc.at[pl.ds(r0 + q * EC, EC)])
        return _
    lax.fori_loop(0, QCH, zero_body, None)

    @pl.when(wid == 0)
    def _():
        pltpu.sync_copy(zb.at[pl.ds(0, 16)], acc.at[pl.ds(SINK, 16)])

    plsc.subcore_barrier()

    def chunk_body(c, _):
        pltpu.sync_copy(ones, acc.at[didx.at[c]], add=True)
        return _
    lax.fori_loop(0, NCHUNK, chunk_body, None)

    plsc.subcore_barrier()

    pltpu.sync_copy(acc.at[pl.ds(r0, RPT)], deg_out.at[pl.ds(r0, RPT)])


def _elu(x):
    return jnp.where(x > 0, x, jnp.exp(jnp.minimum(x, 0.0)) - 1.0)


def _make_prep(d_in, apply_elu):
    def body(x_ref, w_ref, b_ref, deg_ref, g0_ref, a2_ref, b2_ref, a1_ref,
             b1_ref):
        h = jnp.dot(x_ref[...], w_ref[...],
                    preferred_element_type=jnp.float32)
        h = h + jnp.broadcast_to(b_ref[0:1, :], h.shape)
        if apply_elu:
            h = _elu(h)
        deg = deg_ref[...][:, 0:1]
        nb = jax.lax.rsqrt(jnp.clip(deg, 1.0, None))
        nb = jnp.broadcast_to(nb, h.shape)
        g0_ref[...] = nb * h
        a2_ref[...] = (1.0 - ALPHA) * nb * nb
        b2_ref[...] = ALPHA * nb * h
        a1_ref[...] = (1.0 - ALPHA) * nb
        b1_ref[...] = ALPHA * h
    rows = 1024
    grid = NP // rows
    out_sds = jax.ShapeDtypeStruct((NP, HID), jnp.float32)
    out_spec = pl.BlockSpec((rows, HID), lambda i: (i, 0))
    return pl.pallas_call(
        body,
        grid=(grid,),
        in_specs=[
            pl.BlockSpec((rows, d_in), lambda i: (i, 0)),
            pl.BlockSpec((d_in, HID), lambda i: (0, 0)),
            pl.BlockSpec((8, HID), lambda i: (0, 0)),
            pl.BlockSpec((rows, 16), lambda i: (i, 0)),
        ],
        out_specs=[out_spec] * 5,
        out_shape=[out_sds] * 5,
    )


NSLOT = 8
IAHEAD = 6


def _make_appnp(final_elu):
    @functools.partial(
        pl.kernel,
        mesh=_mesh(),
        out_type=jax.ShapeDtypeStruct((NP, HID), jnp.float32),
        scratch_types=[
            pltpu.VMEM((NSLOT, EC), jnp.int32),
            pltpu.VMEM((NSLOT, EC), jnp.int32),
            pltpu.VMEM((4, EC, HID), jnp.float32),
            pltpu.VMEM((UEC, HID), jnp.float32),
            pltpu.VMEM((UEC, HID), jnp.float32),
            pltpu.VMEM((32, HID), jnp.float32),
            pltpu.VMEM_SHARED((GROWS, HID), jnp.float32),
            pltpu.VMEM_SHARED((GROWS, HID), jnp.float32),
            pltpu.SemaphoreType.DMA((4,)),
            pltpu.SemaphoreType.DMA((4,)),
            pltpu.SemaphoreType.DMA((NSLOT,)),
            pltpu.SemaphoreType.DMA((NSLOT,)),
        ],
        compiler_params=_SC_PARAMS,
    )
    def appnp(g0_hbm, a2_hbm, b2_hbm, a1_hbm, b1_hbm, src_hbm, dst_hbm,
              out_hbm, sring, dring, rows, ab, bb, zb, gsp, acc,
              gsem, ssem, isem, jsem):
        wid = lax.axis_index("s")
        r0 = wid * RPT

        def zfill(r, _):
            for j in range(HID // 16):
                zb[r, pl.ds(16 * j, 16)] = jnp.zeros((16,), jnp.float32)
            return _
        lax.fori_loop(0, 32, zfill, None)

        def stage(q, _):
            sl = pl.ds(r0 + q * UEC, UEC)
            stg = rows.at[0, pl.ds(0, UEC)]
            pltpu.sync_copy(g0_hbm.at[sl], stg)
            pltpu.sync_copy(stg, gsp.at[sl])
            return _
        lax.fori_loop(0, UQCH, stage, None)

        def zeroacc(q, _):
            pltpu.sync_copy(zb, acc.at[pl.ds(r0 + q * 32, 32)])
            return _
        lax.fori_loop(0, RPT // 32, zeroacc, None)

        @pl.when(wid == 0)
        def _():
            pltpu.sync_copy(zb.at[pl.ds(0, 16)], gsp.at[pl.ds(SINK, 16)])
            pltpu.sync_copy(zb.at[pl.ds(0, 16)], acc.at[pl.ds(SINK, 16)])

        plsc.subcore_barrier()

        def idx_start(c, m):
            pltpu.async_copy(src_hbm.at[wid, c], sring.at[m], isem.at[m])
            pltpu.async_copy(dst_hbm.at[wid, c], dring.at[m], jsem.at[m])

        def isem_wait(m):
            pltpu.make_async_copy(src_hbm.at[wid, 0], sring.at[m],
                                  isem.at[m]).wait()

        def jsem_wait(m):
            pltpu.make_async_copy(dst_hbm.at[wid, 0], dring.at[m],
                                  jsem.at[m]).wait()

        def gather_start(b, m):
            pltpu.async_copy(gsp.at[sring.at[m]], rows.at[b], gsem.at[b])

        def gather_wait(b, m):
            pltpu.make_async_copy(gsp.at[sring.at[m]], rows.at[b],
                                  gsem.at[b]).wait()

        def scatter_start(b, m):
            pltpu.async_copy(rows.at[b], acc.at[dring.at[m]], ssem.at[b],
                             add=True)

        def scatter_wait(b):
            pltpu.make_async_copy(rows.at[b], acc.at[dring.at[0]],
                                  ssem.at[b]).wait()

        def edge_phase():
            for i in range(IAHEAD):
                idx_start(i, i)
            for i in range(3):
                isem_wait(i)
                gather_start(i, i)

            def chunk_body(c, _):
                for m in range(NSLOT):
                    @pl.when((c % NSLOT) == m)
                    def _():
                        b = m % 4
                        bn = (b + 3) % 4
                        m3 = (m + 3) % NSLOT
                        m6 = (m + IAHEAD) % NSLOT
                        gather_wait(b, m)

                        @pl.when(c >= 1)
                        def _():
                            scatter_wait(bn)

                        @pl.when(c < NCHUNK - 3)
                        def _():
                            isem_wait(m3)
                            gather_start(bn, m3)
                        jsem_wait(m)
                        scatter_start(b, m)

                        @pl.when(c + IAHEAD < NCHUNK)
                        def _():
                            idx_start(c + IAHEAD, m6)
                return _
            lax.fori_loop(0, NCHUNK, chunk_body, None)
            scatter_wait((NCHUNK - 1) % 4)

        def update_phase(a_hbm, b_hbm, dst, elu):
            accv = rows.at[0, pl.ds(0, UEC)]
            gout = rows.at[1, pl.ds(0, UEC)]

            def upd(q, _):
                sl = pl.ds(r0 + q * UEC, UEC)
                pltpu.sync_copy(acc.at[sl], accv)
                pltpu.sync_copy(zb, acc.at[pl.ds(r0 + q * UEC, 32)])
                pltpu.sync_copy(zb, acc.at[pl.ds(r0 + q * UEC + 32, 32)])
                pltpu.sync_copy(a_hbm.at[sl], ab)
                pltpu.sync_copy(b_hbm.at[sl], bb)

                def rowupd(r, _):
                    for j in range(HID // 16):
                        cs = pl.ds(16 * j, 16)
                        v = ab[r, cs] * rows[0, r, cs] + bb[r, cs]
                        if elu:
                            v = jnp.where(
                                v > 0,
                                v,
                                jnp.exp(jnp.minimum(v, 0.0)) - 1.0)
                        rows[1, r, cs] = v
                    return _
                lax.fori_loop(0, UEC, rowupd, None)
                pltpu.sync_copy(gout, dst.at[sl])
                return _
            lax.fori_loop(0, UQCH, upd, None)

        def iter_body(it, _):
            edge_phase()
            plsc.subcore_barrier()
            update_phase(a2_hbm, b2_hbm, gsp, False)
            plsc.subcore_barrier()
            return _
        lax.fori_loop(0, K - 1, iter_body, None)

        edge_phase()
        plsc.subcore_barrier()
        update_phase(a1_hbm, b1_hbm, out_hbm, final_elu)

    return appnp


_appnp_plain = _make_appnp(False)
_appnp_elu = _make_appnp(True)


def kernel(features, edge_index, order_attn, W1, b1, W2, b2):
    del order_attn

    f32 = jnp.float32
    feats = jnp.pad(features.astype(f32), ((0, NP - N), (0, 0)))

    src = edge_index[0].astype(jnp.int32)
    dst = edge_index[1].astype(jnp.int32)
    pad = jnp.full((EPAD - E,), SINK, jnp.int32)
    src3 = jnp.concatenate([src, pad]).reshape(NTILE, NCHUNK, EC)
    dst3 = jnp.concatenate([dst, pad]).reshape(NTILE, NCHUNK, EC)

    deg = _deg_kernel(dst3)

    b1b = jnp.broadcast_to(b1.astype(f32)[None, :], (8, HID))
    b2b = jnp.broadcast_to(b2.astype(f32)[None, :], (8, CLS))

    prep1 = _make_prep(D_IN, False)
    g0, a2, bcoef2, a1, bcoef1 = prep1(feats, W1.astype(f32), b1b, deg)
    x1 = _appnp_plain(g0, a2, bcoef2, a1, bcoef1, src3, dst3)

    prep2 = _make_prep(HID, True)
    g0b, a2b, bcoef2b, a1b, bcoef1b = prep2(x1, W2.astype(f32), b2b, deg)
    x2 = _appnp_elu(g0b, a2b, bcoef2b, a1b, bcoef1b, src3, dst3)

    return x2[:N]
